# speedup vs baseline: 5.8231x; 5.8231x over previous
import functools
import jax
import jax.numpy as jnp
from jax import lax
from jax.experimental import pallas as pl

N = 10000
E = 320000
D_IN = 128
H1, C1 = 8, 8
NUM_CLASSES = 40
ROWS = 1000


def _dense1_body(x_ref, w_ref, am_ref, h_ref, a_ref):
    h = jnp.dot(x_ref[...], w_ref[...], preferred_element_type=jnp.float32)
    h_ref[...] = h
    a_ref[...] = jnp.dot(h, am_ref[...], preferred_element_type=jnp.float32)


def _dense1(x, W1, am1):
    return pl.pallas_call(
        _dense1_body,
        grid=(N // ROWS,),
        in_specs=[
            pl.BlockSpec((ROWS, D_IN), lambda i: (i, 0)),
            pl.BlockSpec((D_IN, H1 * C1), lambda i: (0, 0)),
            pl.BlockSpec((H1 * C1, 2 * H1), lambda i: (0, 0)),
        ],
        out_specs=[
            pl.BlockSpec((ROWS, H1 * C1), lambda i: (i, 0)),
            pl.BlockSpec((ROWS, 2 * H1), lambda i: (i, 0)),
        ],
        out_shape=[
            jax.ShapeDtypeStruct((N, H1 * C1), jnp.float32),
            jax.ShapeDtypeStruct((N, 2 * H1), jnp.float32),
        ],
    )(x, W1, am1)


def _dense2_body(num_ref, den_ref, e8_ref, b1_ref, w2_ref, am2_ref, h2_ref, a2_ref):
    num = num_ref[0] + num_ref[1]
    den = den_ref[0] + den_ref[1]
    den_exp = jnp.dot(den, e8_ref[...], preferred_element_type=jnp.float32)
    out1 = num / (den_exp + 1e-16) + b1_ref[...]
    t = jnp.where(out1 > 0, out1, jnp.exp(jnp.minimum(out1, 0.0)) - 1.0)
    h2 = jnp.dot(t, w2_ref[...], preferred_element_type=jnp.float32)
    h2_ref[...] = h2
    a2_ref[...] = jnp.dot(h2, am2_ref[...], preferred_element_type=jnp.float32)


def _dense2(num1, den1, e8, bias1, W2p, am2):
    C2P = W2p.shape[1]
    return pl.pallas_call(
        _dense2_body,
        grid=(N // ROWS,),
        in_specs=[
            pl.BlockSpec((2, ROWS, H1 * C1), lambda i: (0, i, 0)),
            pl.BlockSpec((2, ROWS, H1), lambda i: (0, i, 0)),
            pl.BlockSpec((H1, H1 * C1), lambda i: (0, 0)),
            pl.BlockSpec((1, H1 * C1), lambda i: (0, 0)),
            pl.BlockSpec((H1 * C1, C2P), lambda i: (0, 0)),
            pl.BlockSpec((C2P, 16), lambda i: (0, 0)),
        ],
        out_specs=[
            pl.BlockSpec((ROWS, C2P), lambda i: (i, 0)),
            pl.BlockSpec((ROWS, 16), lambda i: (i, 0)),
        ],
        out_shape=[
            jax.ShapeDtypeStruct((N, C2P), jnp.float32),
            jax.ShapeDtypeStruct((N, 16), jnp.float32),
        ],
    )(num1, den1, e8, bias1, W2p, am2)


def _final_body(num_ref, den_ref, ones_ref, b2_ref, out_ref):
    num = num_ref[0] + num_ref[1]
    den = den_ref[0] + den_ref[1]
    den_exp = jnp.dot(den, ones_ref[...], preferred_element_type=jnp.float32)
    out2 = num[:, :NUM_CLASSES] / (den_exp + 1e-16) + b2_ref[...]
    m = jnp.max(out2, axis=1, keepdims=True)
    s = out2 - m
    out_ref[...] = s - jnp.log(jnp.sum(jnp.exp(s), axis=1, keepdims=True))


def _final(num2, den2, ones40, bias2):
    C2P = num2.shape[2]
    return pl.pallas_call(
        _final_body,
        grid=(N // ROWS,),
        in_specs=[
            pl.BlockSpec((2, ROWS, C2P), lambda i: (0, i, 0)),
            pl.BlockSpec((2, ROWS, 1), lambda i: (0, i, 0)),
            pl.BlockSpec((1, NUM_CLASSES), lambda i: (0, 0)),
            pl.BlockSpec((1, NUM_CLASSES), lambda i: (0, 0)),
        ],
        out_specs=pl.BlockSpec((ROWS, NUM_CLASSES), lambda i: (i, 0)),
        out_shape=jax.ShapeDtypeStruct((N, NUM_CLASSES), jnp.float32),
    )(num2, den2, ones40, bias2)


def _edge_pass_jax(h, a, src, dst, heads, ch):
    a_src = a[:, :heads]
    a_dst = a[:, heads:2 * heads]
    alpha = a_src[src] + a_dst[dst]
    alpha = jnp.where(alpha >= 0, alpha, 0.2 * alpha)
    ex = jnp.exp(alpha)
    msg = h[src].reshape(E, heads, ch) * ex[:, :, None]
    num = jax.ops.segment_sum(msg.reshape(E, heads * ch), dst, num_segments=N)
    den = jax.ops.segment_sum(ex, dst, num_segments=N)
    return num[None], den[None].astype(jnp.float32)


def kernel(x, edge_index, W1, att_src1, att_dst1, bias1, W2, att_src2, att_dst2, bias2):
    src = edge_index[0]
    dst = edge_index[1]

    am_s1 = jnp.zeros((H1 * C1, H1), jnp.float32)
    am_s1 = am_s1.at[jnp.arange(H1 * C1), jnp.arange(H1 * C1) // C1].set(
        att_src1.reshape(H1 * C1))
    am_d1 = jnp.zeros((H1 * C1, H1), jnp.float32)
    am_d1 = am_d1.at[jnp.arange(H1 * C1), jnp.arange(H1 * C1) // C1].set(
        att_dst1.reshape(H1 * C1))
    am1 = jnp.concatenate([am_s1, am_d1], axis=1)

    e8 = jnp.zeros((H1, H1 * C1), jnp.float32)
    e8 = e8.at[jnp.arange(H1 * C1) // C1, jnp.arange(H1 * C1)].set(1.0)

    C2P = 48
    W2p = jnp.zeros((H1 * C1, C2P), jnp.float32).at[:, :NUM_CLASSES].set(W2)
    am2 = jnp.zeros((C2P, 16), jnp.float32)
    am2 = am2.at[:NUM_CLASSES, 0].set(att_src2.reshape(NUM_CLASSES))
    am2 = am2.at[:NUM_CLASSES, 1].set(att_dst2.reshape(NUM_CLASSES))

    ones40 = jnp.ones((1, NUM_CLASSES), jnp.float32)

    h1, a1 = _dense1(x, W1, am1)
    num1, den1 = _edge_pass_jax(h1, a1, src, dst, H1, C1)
    num1 = jnp.concatenate([num1, jnp.zeros_like(num1)], axis=0)
    den1 = jnp.concatenate([den1, jnp.zeros_like(den1)], axis=0)
    h2, a2 = _dense2(num1, den1, e8, bias1.reshape(1, -1), W2p, am2)
    num2, den2 = _edge_pass_jax(h2, a2, src, dst, 1, C2P)
    num2 = jnp.concatenate([num2, jnp.zeros_like(num2)], axis=0)
    den2 = jnp.concatenate([den2, jnp.zeros_like(den2)], axis=0)
    return _final(num2, den2, ones40, bias2.reshape(1, -1))


# speedup vs baseline: 70.2988x; 12.0724x over previous
import functools
import jax
import jax.numpy as jnp
from jax import lax
from jax.experimental import pallas as pl
from jax.experimental.pallas import tpu as pltpu
from jax.experimental.pallas import tpu_sc as plsc

N = 10000
E = 320000
D_IN = 128
H1, C1 = 8, 8
NUM_CLASSES = 40
ROWS = 1000
C1P = 80
C2P = 48
NP = 10112


def _dense1_body(x_ref, w_ref, ams_ref, amd_ref, h_ref, as_ref, ad_ref):
    h = jnp.dot(x_ref[...], w_ref[...], preferred_element_type=jnp.float32)
    h_ref[...] = h
    as_ref[...] = jnp.dot(h, ams_ref[...], preferred_element_type=jnp.float32)
    ad_ref[...] = jnp.dot(h, amd_ref[...], preferred_element_type=jnp.float32)


def _dense1(x, W1, ams, amd):
    return pl.pallas_call(
        _dense1_body,
        grid=(N // ROWS,),
        in_specs=[
            pl.BlockSpec((ROWS, D_IN), lambda i: (i, 0)),
            pl.BlockSpec((D_IN, H1 * C1), lambda i: (0, 0)),
            pl.BlockSpec((H1 * C1, 16), lambda i: (0, 0)),
            pl.BlockSpec((H1 * C1, 16), lambda i: (0, 0)),
        ],
        out_specs=[
            pl.BlockSpec((ROWS, H1 * C1), lambda i: (i, 0)),
            pl.BlockSpec((ROWS, 16), lambda i: (i, 0)),
            pl.BlockSpec((ROWS, 16), lambda i: (i, 0)),
        ],
        out_shape=[
            jax.ShapeDtypeStruct((N, H1 * C1), jnp.float32),
            jax.ShapeDtypeStruct((N, 16), jnp.float32),
            jax.ShapeDtypeStruct((N, 16), jnp.float32),
        ],
    )(x, W1, ams, amd)


def _dense2_body(acc_ref, e8_ref, b1_ref, w2_ref, ams_ref, amd_ref,
                 h2_ref, as_ref, ad_ref):
    acc = acc_ref[0] + acc_ref[1]
    num = acc[:, :H1 * C1]
    den = acc[:, H1 * C1:H1 * C1 + H1]
    den_exp = jnp.dot(den, e8_ref[...], preferred_element_type=jnp.float32)
    out1 = num / (den_exp + 1e-16) + b1_ref[...]
    t = jnp.where(out1 > 0, out1, jnp.exp(jnp.minimum(out1, 0.0)) - 1.0)
    h2 = jnp.dot(t, w2_ref[...], preferred_element_type=jnp.float32)
    h2_ref[...] = h2
    as_ref[...] = jnp.dot(h2, ams_ref[...], preferred_element_type=jnp.float32)
    ad_ref[...] = jnp.dot(h2, amd_ref[...], preferred_element_type=jnp.float32)


def _dense2(acc1, e8, bias1, W2p, ams2, amd2):
    return pl.pallas_call(
        _dense2_body,
        grid=(N // ROWS,),
        in_specs=[
            pl.BlockSpec((2, ROWS, C1P), lambda i: (0, i, 0)),
            pl.BlockSpec((H1, H1 * C1), lambda i: (0, 0)),
            pl.BlockSpec((1, H1 * C1), lambda i: (0, 0)),
            pl.BlockSpec((H1 * C1, C2P), lambda i: (0, 0)),
            pl.BlockSpec((C2P, 16), lambda i: (0, 0)),
            pl.BlockSpec((C2P, 16), lambda i: (0, 0)),
        ],
        out_specs=[
            pl.BlockSpec((ROWS, C2P), lambda i: (i, 0)),
            pl.BlockSpec((ROWS, 16), lambda i: (i, 0)),
            pl.BlockSpec((ROWS, 16), lambda i: (i, 0)),
        ],
        out_shape=[
            jax.ShapeDtypeStruct((N, C2P), jnp.float32),
            jax.ShapeDtypeStruct((N, 16), jnp.float32),
            jax.ShapeDtypeStruct((N, 16), jnp.float32),
        ],
    )(acc1, e8, bias1, W2p, ams2, amd2)


def _final_body(acc_ref, sel_ref, b2_ref, out_ref):
    acc = acc_ref[0] + acc_ref[1]
    den_exp = jnp.dot(acc, sel_ref[...], preferred_element_type=jnp.float32)
    out2 = acc[:, :NUM_CLASSES] / (den_exp + 1e-16) + b2_ref[...]
    m = jnp.max(out2, axis=1, keepdims=True)
    s = out2 - m
    out_ref[...] = s - jnp.log(jnp.sum(jnp.exp(s), axis=1, keepdims=True))


def _final(acc2, sel, bias2):
    return pl.pallas_call(
        _final_body,
        grid=(N // ROWS,),
        in_specs=[
            pl.BlockSpec((2, ROWS, C2P), lambda i: (0, i, 0)),
            pl.BlockSpec((C2P, NUM_CLASSES), lambda i: (0, 0)),
            pl.BlockSpec((1, NUM_CLASSES), lambda i: (0, 0)),
        ],
        out_specs=pl.BlockSpec((ROWS, NUM_CLASSES), lambda i: (i, 0)),
        out_shape=jax.ShapeDtypeStruct((N, NUM_CLASSES), jnp.float32),
    )(acc2, sel, bias2)


def _edge_pass_sc(src, dst, a_s, a_d, h, zeros, *, heads, chp):
    B = 80
    NW = 32
    EW = E // NW
    ch_h = h.shape[1]
    mesh = plsc.VectorSubcoreMesh(core_axis_name="c", subcore_axis_name="s")

    @functools.partial(
        pl.kernel,
        out_type=jax.ShapeDtypeStruct((2, NP, chp), jnp.float32),
        mesh=mesh,
        compiler_params=pltpu.CompilerParams(use_tc_tiling_on_sc=False),
        scratch_types=[
            pltpu.VMEM((B,), jnp.int32),
            pltpu.VMEM((B,), jnp.int32),
            pltpu.VMEM((B, 16), jnp.float32),
            pltpu.VMEM((B, 16), jnp.float32),
            pltpu.VMEM((B, ch_h), jnp.float32),
            pltpu.VMEM((B, chp), jnp.float32),
            pltpu.VMEM((16,), jnp.float32),
            pltpu.VMEM_SHARED((NP, chp), jnp.float32),
            pltpu.SemaphoreType.DMA,
            pltpu.SemaphoreType.DMA,
            pltpu.SemaphoreType.DMA,
        ],
    )
    def k(src_hbm, dst_hbm, as_hbm, ad_hbm, h_hbm, z_hbm, acc_hbm,
          src_v, dst_v, asr, adr, hr, msg, exb_v, acc_sh, s1, s2, s3):
        cid = lax.axis_index("c")
        sid = lax.axis_index("s")
        wid = cid * 16 + sid
        rpt = NP // 16
        pltpu.sync_copy(z_hbm.at[pl.ds(sid * rpt, rpt)],
                        acc_sh.at[pl.ds(sid * rpt, rpt)])
        plsc.subcore_barrier()

        it = lax.iota(jnp.int32, 16)

        def chunk(i, carry):
            base = wid * EW + i * B
            pltpu.sync_copy(src_hbm.at[pl.ds(base, B)], src_v)
            pltpu.sync_copy(dst_hbm.at[pl.ds(base, B)], dst_v)
            d1 = pltpu.async_copy(as_hbm.at[src_v], asr, s1)
            d2 = pltpu.async_copy(ad_hbm.at[dst_v], adr, s2)
            d3 = pltpu.async_copy(h_hbm.at[src_v], hr, s3)
            d1.wait()
            d2.wait()
            d3.wait()
            for b in range(B):
                al = asr[b, :] + adr[b, :]
                al = jnp.maximum(al, 0.0) + 0.2 * jnp.minimum(al, 0.0)
                exv = jnp.exp(al)
                if heads > 1:
                    for k2 in range(ch_h // 16):
                        e0 = jnp.full((16,), exv[2 * k2], jnp.float32)
                        e1 = jnp.full((16,), exv[2 * k2 + 1], jnp.float32)
                        exb = jnp.where(it < 8, e0, e1)
                        msg[b, 16 * k2:16 * (k2 + 1)] = (
                            hr[b, 16 * k2:16 * (k2 + 1)] * exb)
                    msg[b, ch_h:ch_h + 16] = jnp.where(it < heads, exv, 0.0)
                else:
                    exb = jnp.full((16,), exv[0], jnp.float32)
                    for k2 in range(chp // 16):
                        blk = hr[b, 16 * k2:16 * (k2 + 1)] * exb
                        if 16 * (k2 + 1) == chp:
                            blk = blk + jnp.where(it == 15, exb, 0.0)
                        msg[b, 16 * k2:16 * (k2 + 1)] = blk
            pltpu.sync_copy(msg, acc_sh.at[dst_v], add=True)
            return carry

        lax.fori_loop(0, EW // B, chunk, 0)
        plsc.subcore_barrier()
        rs = sid * rpt
        pltpu.sync_copy(acc_sh.at[pl.ds(rs, rpt)],
                        acc_hbm.at[cid, pl.ds(rs, rpt)])

    return k(src, dst, a_s, a_d, h, zeros)


def kernel(x, edge_index, W1, att_src1, att_dst1, bias1, W2, att_src2, att_dst2, bias2):
    src = edge_index[0]
    dst = edge_index[1]

    rows64 = jnp.arange(H1 * C1)
    ams1 = jnp.zeros((H1 * C1, 16), jnp.float32)
    ams1 = ams1.at[rows64, rows64 // C1].set(att_src1.reshape(H1 * C1))
    amd1 = jnp.zeros((H1 * C1, 16), jnp.float32)
    amd1 = amd1.at[rows64, rows64 // C1].set(att_dst1.reshape(H1 * C1))

    e8 = jnp.zeros((H1, H1 * C1), jnp.float32)
    e8 = e8.at[rows64 // C1, rows64].set(1.0)

    W2p = jnp.zeros((H1 * C1, C2P), jnp.float32).at[:, :NUM_CLASSES].set(W2)
    ams2 = jnp.zeros((C2P, 16), jnp.float32)
    ams2 = ams2.at[:NUM_CLASSES, 0].set(att_src2.reshape(NUM_CLASSES))
    amd2 = jnp.zeros((C2P, 16), jnp.float32)
    amd2 = amd2.at[:NUM_CLASSES, 0].set(att_dst2.reshape(NUM_CLASSES))

    sel = jnp.zeros((C2P, NUM_CLASSES), jnp.float32).at[C2P - 1, :].set(1.0)

    z1 = jnp.zeros((NP, C1P), jnp.float32)
    z2 = jnp.zeros((NP, C2P), jnp.float32)

    h1, a_s1, a_d1 = _dense1(x, W1, ams1, amd1)
    acc1 = _edge_pass_sc(src, dst, a_s1, a_d1, h1, z1, heads=H1, chp=C1P)
    h2, a_s2, a_d2 = _dense2(acc1, e8, bias1.reshape(1, -1), W2p, ams2, amd2)
    acc2 = _edge_pass_sc(src, dst, a_s2, a_d2, h2, z2, heads=1, chp=C2P)
    return _final(acc2, sel, bias2.reshape(1, -1))


# speedup vs baseline: 138.3970x; 1.9687x over previous
import functools
import jax
import jax.numpy as jnp
from jax import lax
from jax.experimental import pallas as pl
from jax.experimental.pallas import tpu as pltpu
from jax.experimental.pallas import tpu_sc as plsc

N = 10000
E = 320000
D_IN = 128
H1, C1 = 8, 8
NUM_CLASSES = 40
ROWS = 1000
C1P = 80
C2P = 48
NP = 10112


def _dense1_body(x_ref, w_ref, ams_ref, amd_ref, h_ref, as_ref, ad_ref):
    h = jnp.dot(x_ref[...], w_ref[...], preferred_element_type=jnp.float32)
    h_ref[...] = h
    as_ref[...] = jnp.dot(h, ams_ref[...], preferred_element_type=jnp.float32)
    ad_ref[...] = jnp.dot(h, amd_ref[...], preferred_element_type=jnp.float32)


def _dense1(x, W1, ams, amd):
    return pl.pallas_call(
        _dense1_body,
        grid=(N // ROWS,),
        in_specs=[
            pl.BlockSpec((ROWS, D_IN), lambda i: (i, 0)),
            pl.BlockSpec((D_IN, H1 * C1), lambda i: (0, 0)),
            pl.BlockSpec((H1 * C1, 16), lambda i: (0, 0)),
            pl.BlockSpec((H1 * C1, 16), lambda i: (0, 0)),
        ],
        out_specs=[
            pl.BlockSpec((ROWS, H1 * C1), lambda i: (i, 0)),
            pl.BlockSpec((ROWS, 16), lambda i: (i, 0)),
            pl.BlockSpec((ROWS, 16), lambda i: (i, 0)),
        ],
        out_shape=[
            jax.ShapeDtypeStruct((N, H1 * C1), jnp.float32),
            jax.ShapeDtypeStruct((N, 16), jnp.float32),
            jax.ShapeDtypeStruct((N, 16), jnp.float32),
        ],
    )(x, W1, ams, amd)


def _dense2_body(acc_ref, e8_ref, b1_ref, w2_ref, ams_ref, amd_ref,
                 h2_ref, as_ref, ad_ref):
    acc = acc_ref[0] + acc_ref[1]
    num = acc[:, :H1 * C1]
    den = acc[:, H1 * C1:H1 * C1 + H1]
    den_exp = jnp.dot(den, e8_ref[...], preferred_element_type=jnp.float32)
    out1 = num / (den_exp + 1e-16) + b1_ref[...]
    t = jnp.where(out1 > 0, out1, jnp.exp(jnp.minimum(out1, 0.0)) - 1.0)
    h2 = jnp.dot(t, w2_ref[...], preferred_element_type=jnp.float32)
    h2_ref[...] = h2
    as_ref[...] = jnp.dot(h2, ams_ref[...], preferred_element_type=jnp.float32)
    ad_ref[...] = jnp.dot(h2, amd_ref[...], preferred_element_type=jnp.float32)


def _dense2(acc1, e8, bias1, W2p, ams2, amd2):
    return pl.pallas_call(
        _dense2_body,
        grid=(N // ROWS,),
        in_specs=[
            pl.BlockSpec((2, ROWS, C1P), lambda i: (0, i, 0)),
            pl.BlockSpec((H1, H1 * C1), lambda i: (0, 0)),
            pl.BlockSpec((1, H1 * C1), lambda i: (0, 0)),
            pl.BlockSpec((H1 * C1, C2P), lambda i: (0, 0)),
            pl.BlockSpec((C2P, 16), lambda i: (0, 0)),
            pl.BlockSpec((C2P, 16), lambda i: (0, 0)),
        ],
        out_specs=[
            pl.BlockSpec((ROWS, C2P), lambda i: (i, 0)),
            pl.BlockSpec((ROWS, 16), lambda i: (i, 0)),
            pl.BlockSpec((ROWS, 16), lambda i: (i, 0)),
        ],
        out_shape=[
            jax.ShapeDtypeStruct((N, C2P), jnp.float32),
            jax.ShapeDtypeStruct((N, 16), jnp.float32),
            jax.ShapeDtypeStruct((N, 16), jnp.float32),
        ],
    )(acc1, e8, bias1, W2p, ams2, amd2)


def _final_body(acc_ref, sel_ref, b2_ref, out_ref):
    acc = acc_ref[0] + acc_ref[1]
    den_exp = jnp.dot(acc, sel_ref[...], preferred_element_type=jnp.float32)
    out2 = acc[:, :NUM_CLASSES] / (den_exp + 1e-16) + b2_ref[...]
    m = jnp.max(out2, axis=1, keepdims=True)
    s = out2 - m
    out_ref[...] = s - jnp.log(jnp.sum(jnp.exp(s), axis=1, keepdims=True))


def _final(acc2, sel, bias2):
    return pl.pallas_call(
        _final_body,
        grid=(N // ROWS,),
        in_specs=[
            pl.BlockSpec((2, ROWS, C2P), lambda i: (0, i, 0)),
            pl.BlockSpec((C2P, NUM_CLASSES), lambda i: (0, 0)),
            pl.BlockSpec((1, NUM_CLASSES), lambda i: (0, 0)),
        ],
        out_specs=pl.BlockSpec((ROWS, NUM_CLASSES), lambda i: (i, 0)),
        out_shape=jax.ShapeDtypeStruct((N, NUM_CLASSES), jnp.float32),
    )(acc2, sel, bias2)


def _edge_pass_sc(src3, dst3, a_s, a_d, h, zeros, *, heads, chp):
    B = 100
    NW = 32
    EW = E // NW
    NCH = EW // B
    ch_h = h.shape[1]
    mesh = plsc.VectorSubcoreMesh(core_axis_name="c", subcore_axis_name="s")

    @functools.partial(
        pl.kernel,
        out_type=jax.ShapeDtypeStruct((2, NP, chp), jnp.float32),
        mesh=mesh,
        compiler_params=pltpu.CompilerParams(use_tc_tiling_on_sc=False),
        scratch_types=[
            pltpu.VMEM((NCH, B), jnp.int32),
            pltpu.VMEM((NCH, B), jnp.int32),
            pltpu.VMEM((2, B, 16), jnp.float32),
            pltpu.VMEM((2, B, 16), jnp.float32),
            pltpu.VMEM((2, B, ch_h), jnp.float32),
            pltpu.VMEM((2, B, chp), jnp.float32),
            pltpu.VMEM_SHARED((NP, chp), jnp.float32),
        ] + [pltpu.SemaphoreType.DMA] * 8,
    )
    def k(src_hbm, dst_hbm, as_hbm, ad_hbm, h_hbm, z_hbm, acc_hbm,
          src_big, dst_big, asr2, adr2, hr2, msg2, acc_sh, *sems):
        cid = lax.axis_index("c")
        sid = lax.axis_index("s")
        wid = cid * 16 + sid
        rpt = NP // 16
        pltpu.sync_copy(src_hbm.at[wid], src_big)
        pltpu.sync_copy(dst_hbm.at[wid], dst_big)
        pltpu.sync_copy(z_hbm.at[pl.ds(sid * rpt, rpt)],
                        acc_sh.at[pl.ds(sid * rpt, rpt)])
        plsc.subcore_barrier()

        it = lax.iota(jnp.int32, 16)
        slots = [
            (asr2.at[j], adr2.at[j], hr2.at[j], msg2.at[j],
             sems[4 * j], sems[4 * j + 1], sems[4 * j + 2], sems[4 * j + 3])
            for j in range(2)
        ]

        def gath_descs(ch, slot):
            asr_, adr_, hr_, _, sa, sb, sh, _ = slot
            return (pltpu.make_async_copy(as_hbm.at[src_big.at[ch]], asr_, sa),
                    pltpu.make_async_copy(ad_hbm.at[dst_big.at[ch]], adr_, sb),
                    pltpu.make_async_copy(h_hbm.at[src_big.at[ch]], hr_, sh))

        def scat_desc(ch, slot):
            msg_, ss = slot[3], slot[7]
            return pltpu.make_async_copy(msg_, acc_sh.at[dst_big.at[ch]], ss)

        def compute(slot):
            asr_, adr_, hr_, msg_ = slot[0], slot[1], slot[2], slot[3]
            for b in range(B):
                al = asr_[b, :] + adr_[b, :]
                al = jnp.maximum(al, 0.0) + 0.2 * jnp.minimum(al, 0.0)
                exv = jnp.exp(al)
                if heads > 1:
                    for k2 in range(ch_h // 16):
                        e0 = jnp.full((16,), exv[2 * k2], jnp.float32)
                        e1 = jnp.full((16,), exv[2 * k2 + 1], jnp.float32)
                        exb = jnp.where(it < 8, e0, e1)
                        msg_[b, 16 * k2:16 * (k2 + 1)] = (
                            hr_[b, 16 * k2:16 * (k2 + 1)] * exb)
                    msg_[b, ch_h:ch_h + 16] = jnp.where(it < heads, exv, 0.0)
                else:
                    exb = jnp.full((16,), exv[0], jnp.float32)
                    for k2 in range(chp // 16):
                        blk = hr_[b, 16 * k2:16 * (k2 + 1)] * exb
                        if 16 * (k2 + 1) == chp:
                            blk = blk + jnp.where(it == 15, exb, 0.0)
                        msg_[b, 16 * k2:16 * (k2 + 1)] = blk

        for j in range(2):
            for desc in gath_descs(j, slots[j]):
                desc.start()

        def body(i2, carry):
            for j in range(2):
                ch = 2 * i2 + j
                slot = slots[j]
                for desc in gath_descs(ch, slot):
                    desc.wait()

                @pl.when(i2 > 0)
                def _():
                    scat_desc(ch, slot).wait()

                compute(slot)
                pltpu.async_copy(slot[3], acc_sh.at[dst_big.at[ch]],
                                 slot[7], add=True)

                @pl.when(i2 < NCH // 2 - 1)
                def _():
                    for desc in gath_descs(ch + 2, slot):
                        desc.start()
            return carry

        lax.fori_loop(0, NCH // 2, body, 0)
        for j in range(2):
            scat_desc(j, slots[j]).wait()
        plsc.subcore_barrier()
        rs = sid * rpt
        pltpu.sync_copy(acc_sh.at[pl.ds(rs, rpt)],
                        acc_hbm.at[cid, pl.ds(rs, rpt)])

    return k(src3, dst3, a_s, a_d, h, zeros)


def kernel(x, edge_index, W1, att_src1, att_dst1, bias1, W2, att_src2, att_dst2, bias2):
    src3 = edge_index[0].reshape(32, (E // 32) // 100, 100)
    dst3 = edge_index[1].reshape(32, (E // 32) // 100, 100)

    rows64 = jnp.arange(H1 * C1)
    ams1 = jnp.zeros((H1 * C1, 16), jnp.float32)
    ams1 = ams1.at[rows64, rows64 // C1].set(att_src1.reshape(H1 * C1))
    amd1 = jnp.zeros((H1 * C1, 16), jnp.float32)
    amd1 = amd1.at[rows64, rows64 // C1].set(att_dst1.reshape(H1 * C1))

    e8 = jnp.zeros((H1, H1 * C1), jnp.float32)
    e8 = e8.at[rows64 // C1, rows64].set(1.0)

    W2p = jnp.zeros((H1 * C1, C2P), jnp.float32).at[:, :NUM_CLASSES].set(W2)
    ams2 = jnp.zeros((C2P, 16), jnp.float32)
    ams2 = ams2.at[:NUM_CLASSES, 0].set(att_src2.reshape(NUM_CLASSES))
    amd2 = jnp.zeros((C2P, 16), jnp.float32)
    amd2 = amd2.at[:NUM_CLASSES, 0].set(att_dst2.reshape(NUM_CLASSES))

    sel = jnp.zeros((C2P, NUM_CLASSES), jnp.float32).at[C2P - 1, :].set(1.0)

    z1 = jnp.zeros((NP, C1P), jnp.float32)
    z2 = jnp.zeros((NP, C2P), jnp.float32)

    h1, a_s1, a_d1 = _dense1(x, W1, ams1, amd1)
    acc1 = _edge_pass_sc(src3, dst3, a_s1, a_d1, h1, z1, heads=H1, chp=C1P)
    h2, a_s2, a_d2 = _dense2(acc1, e8, bias1.reshape(1, -1), W2p, ams2, amd2)
    acc2 = _edge_pass_sc(src3, dst3, a_s2, a_d2, h2, z2, heads=1, chp=C2P)
    return _final(acc2, sel, bias2.reshape(1, -1))


# speedup vs baseline: 163.0254x; 1.1780x over previous
import functools
import jax
import jax.numpy as jnp
from jax import lax
from jax.experimental import pallas as pl
from jax.experimental.pallas import tpu as pltpu
from jax.experimental.pallas import tpu_sc as plsc

N = 10000
E = 320000
D_IN = 128
H1, C1 = 8, 8
NUM_CLASSES = 40
ROWS = 1000
C1P = 80
C2P = 48
NP = 10112


def _dense1_body(x_ref, w_ref, ams_ref, amd_ref, hcat_ref, ad_ref):
    h = jnp.dot(x_ref[...], w_ref[...], preferred_element_type=jnp.float32)
    a_s = jnp.dot(h, ams_ref[...], preferred_element_type=jnp.float32)
    hcat_ref[...] = jnp.concatenate([h, a_s], axis=1)
    ad_ref[...] = jnp.dot(h, amd_ref[...], preferred_element_type=jnp.float32)


def _dense1(x, W1i, ams, amd):
    return pl.pallas_call(
        _dense1_body,
        grid=(N // ROWS,),
        in_specs=[
            pl.BlockSpec((ROWS, D_IN), lambda i: (i, 0)),
            pl.BlockSpec((D_IN, H1 * C1), lambda i: (0, 0)),
            pl.BlockSpec((H1 * C1, 16), lambda i: (0, 0)),
            pl.BlockSpec((H1 * C1, 16), lambda i: (0, 0)),
        ],
        out_specs=[
            pl.BlockSpec((ROWS, H1 * C1 + 16), lambda i: (i, 0)),
            pl.BlockSpec((ROWS, 16), lambda i: (i, 0)),
        ],
        out_shape=[
            jax.ShapeDtypeStruct((N, H1 * C1 + 16), jnp.float32),
            jax.ShapeDtypeStruct((N, 16), jnp.float32),
        ],
    )(x, W1i, ams, amd)


def _dense2_body(acc_ref, e8_ref, b1_ref, w2_ref, ams_ref, amd_ref,
                 hcat_ref, ad_ref):
    acc = acc_ref[0] + acc_ref[1]
    num = acc[:, :H1 * C1]
    den = acc[:, H1 * C1:H1 * C1 + H1]
    den_exp = jnp.dot(den, e8_ref[...], preferred_element_type=jnp.float32)
    out1 = num / (den_exp + 1e-16) + b1_ref[...]
    t = jnp.where(out1 > 0, out1, jnp.exp(jnp.minimum(out1, 0.0)) - 1.0)
    h2 = jnp.dot(t, w2_ref[...], preferred_element_type=jnp.float32)
    a_s = jnp.dot(h2, ams_ref[...], preferred_element_type=jnp.float32)
    hcat_ref[...] = jnp.concatenate([h2, a_s], axis=1)
    ad_ref[...] = jnp.dot(h2, amd_ref[...], preferred_element_type=jnp.float32)


def _dense2(acc1, e8, bias1, W2p, ams2, amd2):
    return pl.pallas_call(
        _dense2_body,
        grid=(N // ROWS,),
        in_specs=[
            pl.BlockSpec((2, ROWS, C1P), lambda i: (0, i, 0)),
            pl.BlockSpec((H1, H1 * C1), lambda i: (0, 0)),
            pl.BlockSpec((1, H1 * C1), lambda i: (0, 0)),
            pl.BlockSpec((H1 * C1, C2P), lambda i: (0, 0)),
            pl.BlockSpec((C2P, 16), lambda i: (0, 0)),
            pl.BlockSpec((C2P, 16), lambda i: (0, 0)),
        ],
        out_specs=[
            pl.BlockSpec((ROWS, C2P + 16), lambda i: (i, 0)),
            pl.BlockSpec((ROWS, 16), lambda i: (i, 0)),
        ],
        out_shape=[
            jax.ShapeDtypeStruct((N, C2P + 16), jnp.float32),
            jax.ShapeDtypeStruct((N, 16), jnp.float32),
        ],
    )(acc1, e8, bias1, W2p, ams2, amd2)


def _final_body(acc_ref, sel_ref, b2_ref, out_ref):
    acc = acc_ref[0] + acc_ref[1]
    den_exp = jnp.dot(acc, sel_ref[...], preferred_element_type=jnp.float32)
    out2 = acc[:, :NUM_CLASSES] / (den_exp + 1e-16) + b2_ref[...]
    m = jnp.max(out2, axis=1, keepdims=True)
    s = out2 - m
    out_ref[...] = s - jnp.log(jnp.sum(jnp.exp(s), axis=1, keepdims=True))


def _final(acc2, sel, bias2):
    return pl.pallas_call(
        _final_body,
        grid=(N // ROWS,),
        in_specs=[
            pl.BlockSpec((2, ROWS, C2P), lambda i: (0, i, 0)),
            pl.BlockSpec((C2P, NUM_CLASSES), lambda i: (0, 0)),
            pl.BlockSpec((1, NUM_CLASSES), lambda i: (0, 0)),
        ],
        out_specs=pl.BlockSpec((ROWS, NUM_CLASSES), lambda i: (i, 0)),
        out_shape=jax.ShapeDtypeStruct((N, NUM_CLASSES), jnp.float32),
    )(acc2, sel, bias2)


def _edge_pass_sc(src3, dst3, a_d, hcat, zeros, *, heads, chp):
    B = 100
    NW = 32
    EW = E // NW
    NCH = EW // B
    ch_h = hcat.shape[1]
    nh = (ch_h - 16) // 16
    mesh = plsc.VectorSubcoreMesh(core_axis_name="c", subcore_axis_name="s")

    @functools.partial(
        pl.kernel,
        out_type=jax.ShapeDtypeStruct((2, NP, chp), jnp.float32),
        mesh=mesh,
        compiler_params=pltpu.CompilerParams(use_tc_tiling_on_sc=False),
        scratch_types=[
            pltpu.VMEM((NCH, B), jnp.int32),
            pltpu.VMEM((NCH, B), jnp.int32),
            pltpu.VMEM((2, B, 16), jnp.float32),
            pltpu.VMEM((2, B, ch_h), jnp.float32),
            pltpu.VMEM((2, B, chp), jnp.float32),
            pltpu.VMEM_SHARED((NP, chp), jnp.float32),
        ] + [pltpu.SemaphoreType.DMA] * 6,
    )
    def k(src_hbm, dst_hbm, ad_hbm, h_hbm, z_hbm, acc_hbm,
          src_big, dst_big, adr2, hr2, msg2, acc_sh, *sems):
        cid = lax.axis_index("c")
        sid = lax.axis_index("s")
        wid = cid * 16 + sid
        rpt = NP // 16
        pltpu.sync_copy(src_hbm.at[wid], src_big)
        pltpu.sync_copy(dst_hbm.at[wid], dst_big)
        pltpu.sync_copy(z_hbm.at[pl.ds(sid * rpt, rpt)],
                        acc_sh.at[pl.ds(sid * rpt, rpt)])
        plsc.subcore_barrier()

        it = lax.iota(jnp.int32, 16)
        slots = [
            (adr2.at[j], hr2.at[j], msg2.at[j],
             sems[3 * j], sems[3 * j + 1], sems[3 * j + 2])
            for j in range(2)
        ]

        def gath_descs(ch, slot):
            adr_, hr_, _, sa, sh, _ = slot
            return (pltpu.make_async_copy(ad_hbm.at[dst_big.at[ch]], adr_, sa),
                    pltpu.make_async_copy(h_hbm.at[src_big.at[ch]], hr_, sh))

        def scat_desc(ch, slot):
            msg_, ss = slot[2], slot[5]
            return pltpu.make_async_copy(msg_, acc_sh.at[dst_big.at[ch]], ss)

        def compute(slot):
            adr_, hr_, msg_ = slot[0], slot[1], slot[2]
            for b in range(B):
                al = hr_[b, ch_h - 16:ch_h] + adr_[b, :]
                al = jnp.maximum(al, 0.0) + 0.2 * jnp.minimum(al, 0.0)
                exv = jnp.exp(al)
                for k2 in range(nh):
                    blk = hr_[b, 16 * k2:16 * (k2 + 1)] * exv
                    if heads == 1 and k2 == nh - 1:
                        blk = blk + jnp.where(it == 15, exv, 0.0)
                    msg_[b, 16 * k2:16 * (k2 + 1)] = blk
                if heads > 1:
                    msg_[b, 16 * nh:16 * (nh + 1)] = jnp.where(
                        it < heads, exv, 0.0)

        for j in range(2):
            for desc in gath_descs(j, slots[j]):
                desc.start()

        def body(i2, carry):
            for j in range(2):
                ch = 2 * i2 + j
                slot = slots[j]
                for desc in gath_descs(ch, slot):
                    desc.wait()

                @pl.when(i2 > 0)
                def _():
                    scat_desc(ch, slot).wait()

                compute(slot)
                pltpu.async_copy(slot[2], acc_sh.at[dst_big.at[ch]],
                                 slot[5], add=True)

                @pl.when(i2 < NCH // 2 - 1)
                def _():
                    for desc in gath_descs(ch + 2, slot):
                        desc.start()
            return carry

        lax.fori_loop(0, NCH // 2, body, 0)
        for j in range(2):
            scat_desc(j, slots[j]).wait()
        plsc.subcore_barrier()
        rs = sid * rpt
        pltpu.sync_copy(acc_sh.at[pl.ds(rs, rpt)],
                        acc_hbm.at[cid, pl.ds(rs, rpt)])

    return k(src3, dst3, a_d, hcat, zeros)


def kernel(x, edge_index, W1, att_src1, att_dst1, bias1, W2, att_src2, att_dst2, bias2):
    src3 = edge_index[0].reshape(32, (E // 32) // 100, 100)
    dst3 = edge_index[1].reshape(32, (E // 32) // 100, 100)

    rows64 = jnp.arange(H1 * C1)
    perm = (rows64 % H1) * C1 + rows64 // H1
    W1i = W1[:, perm]
    bias1i = bias1[perm]

    g64 = rows64 % H1
    c64 = rows64 // H1
    ams1 = jnp.zeros((H1 * C1, 16), jnp.float32)
    ams1 = ams1.at[rows64, g64].set(att_src1[0, g64, c64])
    ams1 = ams1.at[rows64, g64 + 8].set(att_src1[0, g64, c64])
    amd1 = jnp.zeros((H1 * C1, 16), jnp.float32)
    amd1 = amd1.at[rows64, g64].set(att_dst1[0, g64, c64])
    amd1 = amd1.at[rows64, g64 + 8].set(att_dst1[0, g64, c64])

    e8 = jnp.zeros((H1, H1 * C1), jnp.float32)
    e8 = e8.at[g64, rows64].set(1.0)

    W2p = jnp.zeros((H1 * C1, C2P), jnp.float32).at[:, :NUM_CLASSES].set(W2[perm, :])
    ams2 = jnp.zeros((C2P, 16), jnp.float32)
    ams2 = ams2.at[:NUM_CLASSES, :].set(att_src2.reshape(NUM_CLASSES, 1))
    amd2 = jnp.zeros((C2P, 16), jnp.float32)
    amd2 = amd2.at[:NUM_CLASSES, :].set(att_dst2.reshape(NUM_CLASSES, 1))

    sel = jnp.zeros((C2P, NUM_CLASSES), jnp.float32).at[C2P - 1, :].set(1.0)

    z1 = jnp.zeros((NP, C1P), jnp.float32)
    z2 = jnp.zeros((NP, C2P), jnp.float32)

    hcat1, a_d1 = _dense1(x, W1i, ams1, amd1)
    acc1 = _edge_pass_sc(src3, dst3, a_d1, hcat1, z1, heads=H1, chp=C1P)
    hcat2, a_d2 = _dense2(acc1, e8, bias1i.reshape(1, -1), W2p, ams2, amd2)
    acc2 = _edge_pass_sc(src3, dst3, a_d2, hcat2, z2, heads=1, chp=C2P)
    return _final(acc2, sel, bias2.reshape(1, -1))


# speedup vs baseline: 168.3912x; 1.0329x over previous
import functools
import jax
import jax.numpy as jnp
from jax import lax
from jax.experimental import pallas as pl
from jax.experimental.pallas import tpu as pltpu
from jax.experimental.pallas import tpu_sc as plsc

N = 10000
E = 320000
D_IN = 128
H1, C1 = 8, 8
NUM_CLASSES = 40
ROWS = 5000
C1P = 80
C2P = 48
NP = 10112


def _dense1_body(x_ref, w_ref, ams_ref, amd_ref, hcat_ref, ad_ref):
    h = jnp.dot(x_ref[...], w_ref[...], preferred_element_type=jnp.float32)
    a_s = jnp.dot(h, ams_ref[...], preferred_element_type=jnp.float32)
    hcat_ref[...] = jnp.concatenate([h, a_s], axis=1)
    ad_ref[...] = jnp.dot(h, amd_ref[...], preferred_element_type=jnp.float32)


def _dense1(x, W1i, ams, amd):
    return pl.pallas_call(
        _dense1_body,
        grid=(N // ROWS,),
        in_specs=[
            pl.BlockSpec((ROWS, D_IN), lambda i: (i, 0)),
            pl.BlockSpec((D_IN, H1 * C1), lambda i: (0, 0)),
            pl.BlockSpec((H1 * C1, 16), lambda i: (0, 0)),
            pl.BlockSpec((H1 * C1, 16), lambda i: (0, 0)),
        ],
        out_specs=[
            pl.BlockSpec((ROWS, H1 * C1 + 16), lambda i: (i, 0)),
            pl.BlockSpec((ROWS, 16), lambda i: (i, 0)),
        ],
        out_shape=[
            jax.ShapeDtypeStruct((N, H1 * C1 + 16), jnp.float32),
            jax.ShapeDtypeStruct((N, 16), jnp.float32),
        ],
    )(x, W1i, ams, amd)


def _dense2_body(acc_ref, e8_ref, b1_ref, w2_ref, ams_ref, amd_ref,
                 hcat_ref, ad_ref):
    acc = acc_ref[0] + acc_ref[1]
    num = acc[:, :H1 * C1]
    den = acc[:, H1 * C1:H1 * C1 + H1]
    den_exp = jnp.dot(den, e8_ref[...], preferred_element_type=jnp.float32)
    out1 = num / (den_exp + 1e-16) + b1_ref[...]
    t = jnp.where(out1 > 0, out1, jnp.exp(jnp.minimum(out1, 0.0)) - 1.0)
    h2 = jnp.dot(t, w2_ref[...], preferred_element_type=jnp.float32)
    a_s = jnp.dot(h2, ams_ref[...], preferred_element_type=jnp.float32)
    hcat_ref[...] = jnp.concatenate([h2, a_s], axis=1)
    ad_ref[...] = jnp.dot(h2, amd_ref[...], preferred_element_type=jnp.float32)


def _dense2(acc1, e8, bias1, W2p, ams2, amd2):
    return pl.pallas_call(
        _dense2_body,
        grid=(N // ROWS,),
        in_specs=[
            pl.BlockSpec((2, ROWS, C1P), lambda i: (0, i, 0)),
            pl.BlockSpec((H1, H1 * C1), lambda i: (0, 0)),
            pl.BlockSpec((1, H1 * C1), lambda i: (0, 0)),
            pl.BlockSpec((H1 * C1, C2P), lambda i: (0, 0)),
            pl.BlockSpec((C2P, 16), lambda i: (0, 0)),
            pl.BlockSpec((C2P, 16), lambda i: (0, 0)),
        ],
        out_specs=[
            pl.BlockSpec((ROWS, C2P + 16), lambda i: (i, 0)),
            pl.BlockSpec((ROWS, 16), lambda i: (i, 0)),
        ],
        out_shape=[
            jax.ShapeDtypeStruct((N, C2P + 16), jnp.float32),
            jax.ShapeDtypeStruct((N, 16), jnp.float32),
        ],
    )(acc1, e8, bias1, W2p, ams2, amd2)


def _final_body(acc_ref, sel_ref, b2_ref, out_ref):
    acc = acc_ref[0] + acc_ref[1]
    den_exp = jnp.dot(acc, sel_ref[...], preferred_element_type=jnp.float32)
    out2 = acc[:, :NUM_CLASSES] / (den_exp + 1e-16) + b2_ref[...]
    m = jnp.max(out2, axis=1, keepdims=True)
    s = out2 - m
    out_ref[...] = s - jnp.log(jnp.sum(jnp.exp(s), axis=1, keepdims=True))


def _final(acc2, sel, bias2):
    return pl.pallas_call(
        _final_body,
        grid=(N // ROWS,),
        in_specs=[
            pl.BlockSpec((2, ROWS, C2P), lambda i: (0, i, 0)),
            pl.BlockSpec((C2P, NUM_CLASSES), lambda i: (0, 0)),
            pl.BlockSpec((1, NUM_CLASSES), lambda i: (0, 0)),
        ],
        out_specs=pl.BlockSpec((ROWS, NUM_CLASSES), lambda i: (i, 0)),
        out_shape=jax.ShapeDtypeStruct((N, NUM_CLASSES), jnp.float32),
    )(acc2, sel, bias2)


def _edge_pass_sc(src3, dst3, a_d, hcat, zeros, *, heads, chp):
    B = 100
    NW = 32
    EW = E // NW
    NCH = EW // B
    ch_h = hcat.shape[1]
    nh = (ch_h - 16) // 16
    mesh = plsc.VectorSubcoreMesh(core_axis_name="c", subcore_axis_name="s")

    @functools.partial(
        pl.kernel,
        out_type=jax.ShapeDtypeStruct((2, NP, chp), jnp.float32),
        mesh=mesh,
        compiler_params=pltpu.CompilerParams(use_tc_tiling_on_sc=False),
        scratch_types=[
            pltpu.VMEM((NCH, B), jnp.int32),
            pltpu.VMEM((NCH, B), jnp.int32),
            pltpu.VMEM((2, B, 16), jnp.float32),
            pltpu.VMEM((2, B, ch_h), jnp.float32),
            pltpu.VMEM((2, B, chp), jnp.float32),
            pltpu.VMEM_SHARED((NP, chp), jnp.float32),
        ] + [pltpu.SemaphoreType.DMA] * 6,
    )
    def k(src_hbm, dst_hbm, ad_hbm, h_hbm, z_hbm, acc_hbm,
          src_big, dst_big, adr2, hr2, msg2, acc_sh, *sems):
        cid = lax.axis_index("c")
        sid = lax.axis_index("s")
        wid = cid * 16 + sid
        rpt = NP // 16
        pltpu.sync_copy(src_hbm.at[wid], src_big)
        pltpu.sync_copy(dst_hbm.at[wid], dst_big)
        pltpu.sync_copy(z_hbm, acc_sh.at[pl.ds(sid * rpt, rpt)])
        plsc.subcore_barrier()

        it = lax.iota(jnp.int32, 16)
        slots = [
            (adr2.at[j], hr2.at[j], msg2.at[j],
             sems[3 * j], sems[3 * j + 1], sems[3 * j + 2])
            for j in range(2)
        ]

        def gath_descs(ch, slot):
            adr_, hr_, _, sa, sh, _ = slot
            return (pltpu.make_async_copy(ad_hbm.at[dst_big.at[ch]], adr_, sa),
                    pltpu.make_async_copy(h_hbm.at[src_big.at[ch]], hr_, sh))

        def scat_desc(ch, slot):
            msg_, ss = slot[2], slot[5]
            return pltpu.make_async_copy(msg_, acc_sh.at[dst_big.at[ch]], ss)

        def compute(slot):
            adr_, hr_, msg_ = slot[0], slot[1], slot[2]
            for b in range(B):
                al = hr_[b, ch_h - 16:ch_h] + adr_[b, :]
                al = jnp.maximum(al, 0.0) + 0.2 * jnp.minimum(al, 0.0)
                exv = jnp.exp(al)
                for k2 in range(nh):
                    blk = hr_[b, 16 * k2:16 * (k2 + 1)] * exv
                    if heads == 1 and k2 == nh - 1:
                        blk = blk + jnp.where(it == 15, exv, 0.0)
                    msg_[b, 16 * k2:16 * (k2 + 1)] = blk
                if heads > 1:
                    msg_[b, 16 * nh:16 * (nh + 1)] = jnp.where(
                        it < heads, exv, 0.0)

        for j in range(2):
            for desc in gath_descs(j, slots[j]):
                desc.start()

        def body(i2, carry):
            for j in range(2):
                ch = 2 * i2 + j
                slot = slots[j]
                for desc in gath_descs(ch, slot):
                    desc.wait()

                @pl.when(i2 > 0)
                def _():
                    scat_desc(ch, slot).wait()

                compute(slot)
                pltpu.async_copy(slot[2], acc_sh.at[dst_big.at[ch]],
                                 slot[5], add=True)

                @pl.when(i2 < NCH // 2 - 1)
                def _():
                    for desc in gath_descs(ch + 2, slot):
                        desc.start()
            return carry

        lax.fori_loop(0, NCH // 2, body, 0)
        for j in range(2):
            scat_desc(j, slots[j]).wait()
        plsc.subcore_barrier()
        rs = sid * rpt
        pltpu.sync_copy(acc_sh.at[pl.ds(rs, rpt)],
                        acc_hbm.at[cid, pl.ds(rs, rpt)])

    return k(src3, dst3, a_d, hcat, zeros)


def kernel(x, edge_index, W1, att_src1, att_dst1, bias1, W2, att_src2, att_dst2, bias2):
    src3 = edge_index[0].reshape(32, (E // 32) // 100, 100)
    dst3 = edge_index[1].reshape(32, (E // 32) // 100, 100)

    rows64 = jnp.arange(H1 * C1)
    perm = (rows64 % H1) * C1 + rows64 // H1
    W1i = W1[:, perm]
    bias1i = bias1[perm]

    g64 = rows64 % H1
    c64 = rows64 // H1
    ams1 = jnp.zeros((H1 * C1, 16), jnp.float32)
    ams1 = ams1.at[rows64, g64].set(att_src1[0, g64, c64])
    ams1 = ams1.at[rows64, g64 + 8].set(att_src1[0, g64, c64])
    amd1 = jnp.zeros((H1 * C1, 16), jnp.float32)
    amd1 = amd1.at[rows64, g64].set(att_dst1[0, g64, c64])
    amd1 = amd1.at[rows64, g64 + 8].set(att_dst1[0, g64, c64])

    e8 = jnp.zeros((H1, H1 * C1), jnp.float32)
    e8 = e8.at[g64, rows64].set(1.0)

    W2p = jnp.zeros((H1 * C1, C2P), jnp.float32).at[:, :NUM_CLASSES].set(W2[perm, :])
    ams2 = jnp.zeros((C2P, 16), jnp.float32)
    ams2 = ams2.at[:NUM_CLASSES, :].set(att_src2.reshape(NUM_CLASSES, 1))
    amd2 = jnp.zeros((C2P, 16), jnp.float32)
    amd2 = amd2.at[:NUM_CLASSES, :].set(att_dst2.reshape(NUM_CLASSES, 1))

    sel = jnp.zeros((C2P, NUM_CLASSES), jnp.float32).at[C2P - 1, :].set(1.0)

    z1 = jnp.zeros((NP // 16, C1P), jnp.float32)
    z2 = jnp.zeros((NP // 16, C2P), jnp.float32)

    hcat1, a_d1 = _dense1(x, W1i, ams1, amd1)
    acc1 = _edge_pass_sc(src3, dst3, a_d1, hcat1, z1, heads=H1, chp=C1P)
    hcat2, a_d2 = _dense2(acc1, e8, bias1i.reshape(1, -1), W2p, ams2, amd2)
    acc2 = _edge_pass_sc(src3, dst3, a_d2, hcat2, z2, heads=1, chp=C2P)
    return _final(acc2, sel, bias2.reshape(1, -1))
